# baseline (speedup 1.0000x reference)
import functools

import jax
import jax.numpy as jnp
from jax import lax
from jax.experimental import pallas as pl
from jax.experimental.pallas import tpu as pltpu
from jax.experimental.pallas import tpu_sc as plsc

B = 4096
D = 16
L_I = 200
L_G = 50
NW = 32
BW = B // NW
RI = 2 * B
RG = B // 2


def _sc_gather_kernel(user_idx, item_idx, hm2, hg2,
                      user_emb, item_emb, genre_emb,
                      user_linf, item_linf, genre_linf,
                      user_e, item_e, user_l, item_l,
                      hist_i_sum, hist_g_sum, hist_i_linp, hist_g_linp,
                      idx_v, rows128_v, lin128_v, idsA_v, idsB_v,
                      growsA_v, growsB_v, linA_v, linB_v,
                      sums_v, linp_v, sem, semA, semB):
  wid = lax.axis_index("s") * 2 + lax.axis_index("c")
  base = wid * BW

  pltpu.sync_copy(user_idx.at[pl.ds(base, BW)], idx_v)
  pltpu.async_copy(user_emb.at[idx_v], rows128_v, sem).wait()
  pltpu.sync_copy(rows128_v, user_e.at[pl.ds(base, BW)])
  pltpu.async_copy(user_linf.at[idx_v], lin128_v, sem).wait()
  pltpu.sync_copy(lin128_v, user_l.at[pl.ds(base, BW)])

  pltpu.sync_copy(item_idx.at[pl.ds(base, BW)], idx_v)
  pltpu.async_copy(item_emb.at[idx_v], rows128_v, sem).wait()
  pltpu.sync_copy(rows128_v, item_e.at[pl.ds(base, BW)])
  pltpu.async_copy(item_linf.at[idx_v], lin128_v, sem).wait()
  pltpu.sync_copy(lin128_v, item_l.at[pl.ds(base, BW)])

  z16 = jnp.zeros((16,), jnp.float32)
  lanes = lax.iota(jnp.int32, 16)

  def hist_pass(ids2_hbm, table, linf, sum_out, linp_out, n_chunks,
                elems_per_chunk, row_base, genre_mode):
    def copies(ids_s, grows_s, lin_s, sem_s):
      cs = []
      for j in range(16):
        cs.append(pltpu.make_async_copy(
            table.at[ids_s.at[j]], grows_s.at[pl.ds(j * 112, 112)], sem_s))
        cs.append(pltpu.make_async_copy(
            linf.at[ids_s.at[j]], lin_s.at[j], sem_s))
      return cs

    def fire(ch, ids_s, grows_s, lin_s, sem_s):
      pltpu.sync_copy(ids2_hbm.at[pl.ds(row_base + ch * 16, 16)], ids_s)
      for c in copies(ids_s, grows_s, lin_s, sem_s):
        c.start()

    def drain(ids_s, grows_s, lin_s, sem_s):
      for c in copies(ids_s, grows_s, lin_s, sem_s):
        c.wait()

    def process(ch, grows_s, lin_s):
      if genre_mode:
        run_len = 50
        start_of = lambda e: (e // 2) * 112 + (e % 2) * 50
      else:
        run_len = 224
        start_of = lambda e: e * 224
      n4 = run_len // 4
      rem = run_len - n4 * 4
      def elem_body(e, _):
        start = start_of(e)
        def add_body(l, accs):
          b = start + 4 * l
          return (accs[0] + grows_s[b], accs[1] + grows_s[b + 1],
                  accs[2] + grows_s[b + 2], accs[3] + grows_s[b + 3])
        a = lax.fori_loop(0, n4, add_body, (z16, z16, z16, z16), unroll=4)
        acc = (a[0] + a[1]) + (a[2] + a[3])
        for r in range(rem):
          acc = acc + grows_s[start + n4 * 4 + r]
        sums_v[ch * elems_per_chunk + e] = acc
        return 0
      lax.fori_loop(0, elems_per_chunk, elem_body, 0)

      if not genre_mode:
        def lin_body(e, _):
          acc = z16
          for half in range(2):
            for k in range(7):
              acc = acc + lin_s[2 * e + half, pl.ds(k * 16, 16)]
          linp_v[ch * elems_per_chunk + e] = acc
          return 0
        lax.fori_loop(0, elems_per_chunk, lin_body, 0)
      else:
        def lin_body(j, _):
          v = [lin_s[j, pl.ds(k * 16, 16)] for k in range(7)]
          zf = jnp.zeros((16,), jnp.float32)
          sa = v[0] + v[1] + v[2] + jnp.where(lanes < 2, v[3], zf)
          sb = jnp.where(lanes >= 2, v[3], zf) + v[4] + v[5] + v[6]
          linp_v[ch * 32 + 2 * j] = sa
          linp_v[ch * 32 + 2 * j + 1] = sb
          return 0
        lax.fori_loop(0, 16, lin_body, 0)

    fire(0, idsA_v, growsA_v, linA_v, semA)
    def pair_body(p, _):
      ch0 = 2 * p
      drain(idsA_v, growsA_v, linA_v, semA)
      fire(ch0 + 1, idsB_v, growsB_v, linB_v, semB)
      process(ch0, growsA_v, linA_v)
      drain(idsB_v, growsB_v, linB_v, semB)
      @pl.when(ch0 + 2 < n_chunks)
      def _():
        fire(ch0 + 2, idsA_v, growsA_v, linA_v, semA)
      process(ch0 + 1, growsB_v, linB_v)
      return 0
    lax.fori_loop(0, n_chunks // 2, pair_body, 0)
    pltpu.sync_copy(sums_v, sum_out.at[pl.ds(base, BW)])
    pltpu.sync_copy(linp_v, linp_out.at[pl.ds(base, BW)])

  hist_pass(hm2, item_emb, item_linf, hist_i_sum, hist_i_linp,
            n_chunks=16, elems_per_chunk=8, row_base=wid * 256,
            genre_mode=False)
  hist_pass(hg2, genre_emb, genre_linf, hist_g_sum, hist_g_linp,
            n_chunks=4, elems_per_chunk=32, row_base=wid * 64,
            genre_mode=True)


@jax.jit
def _sc_gather(user_idx, item_idx, hm2, hg2, user_emb, item_emb, genre_emb,
               user_linf, item_linf, genre_linf):
  f32 = jnp.float32
  out_type = (
      jax.ShapeDtypeStruct((B, D), f32),
      jax.ShapeDtypeStruct((B, D), f32),
      jax.ShapeDtypeStruct((B,), f32),
      jax.ShapeDtypeStruct((B,), f32),
      jax.ShapeDtypeStruct((B, D), f32),
      jax.ShapeDtypeStruct((B, D), f32),
      jax.ShapeDtypeStruct((B, D), f32),
      jax.ShapeDtypeStruct((B, D), f32),
  )
  scratch = [
      pltpu.VMEM((BW,), jnp.int32),
      pltpu.VMEM((BW, D), f32),
      pltpu.VMEM((BW,), f32),
      pltpu.VMEM((16, 112), jnp.int32),
      pltpu.VMEM((16, 112), jnp.int32),
      pltpu.VMEM((1792, D), f32),
      pltpu.VMEM((1792, D), f32),
      pltpu.VMEM((16, 112), f32),
      pltpu.VMEM((16, 112), f32),
      pltpu.VMEM((BW, D), f32),
      pltpu.VMEM((BW, D), f32),
      pltpu.SemaphoreType.DMA,
      pltpu.SemaphoreType.DMA,
      pltpu.SemaphoreType.DMA,
  ]
  mesh = plsc.VectorSubcoreMesh(core_axis_name="c", subcore_axis_name="s")
  return pl.kernel(_sc_gather_kernel, out_type=out_type, mesh=mesh,
                   scratch_types=scratch,
                   compiler_params=pltpu.CompilerParams(
                       use_tc_tiling_on_sc=False))(
      user_idx, item_idx, hm2, hg2, user_emb, item_emb, genre_emb,
      user_linf, item_linf, genre_linf)


def _tc_dense_kernel(user_e, item_e, hist_i_sum, hist_g_sum,
                     user_l, item_l, hist_i_linp, hist_g_linp,
                     hm_ids, hg_ids, stat_ids, nums,
                     stat_emb_W, stat_lin_W,
                     W1, b1, s1, be1, W2, b2, s2, be2, Wout, bout,
                     out_ref):
  f32 = jnp.float32
  cnt_i = jnp.sum((hm_ids[...] != 0).astype(f32), axis=1, keepdims=True)
  cnt_g = jnp.sum((hg_ids[...] != 0).astype(f32), axis=1, keepdims=True)
  inv_i = 1.0 / (cnt_i + 1e-8)
  inv_g = 1.0 / (cnt_g + 1e-8)

  mean_i = hist_i_sum[...] * inv_i
  mean_g = hist_g_sum[...] * inv_g

  lin = (user_l[...] + item_l[...]
         + jnp.sum(hist_i_linp[...], axis=1, keepdims=True) * inv_i
         + jnp.sum(hist_g_linp[...], axis=1, keepdims=True) * inv_g)

  iota100 = lax.broadcasted_iota(jnp.int32, (B, 100), 1)
  sids = stat_ids[...]
  embs = [user_e[...], item_e[...]]
  for j in range(5):
    oh = (iota100 == sids[:, j][:, None]).astype(f32)
    embs.append(jnp.dot(oh, stat_emb_W[j], preferred_element_type=f32))
    lin = lin + jnp.dot(oh, stat_lin_W[j], preferred_element_type=f32)
  embs.append(mean_i)
  embs.append(mean_g)

  S = embs[0]
  Q = embs[0] * embs[0]
  for e in embs[1:]:
    S = S + e
    Q = Q + e * e
  fm = 0.5 * jnp.sum(S * S - Q, axis=1, keepdims=True)

  h1 = jnp.zeros((B, 128), f32)
  for f in range(9):
    h1 = h1 + jnp.dot(embs[f], W1[f * 16:(f + 1) * 16, :],
                      preferred_element_type=f32)
  nm = nums[...]
  for t in range(3):
    h1 = h1 + nm[:, t][:, None] * W1[144 + t, :][None, :]
  h1 = jnp.maximum((h1 + b1[...][None, :]) * s1[...][None, :]
                   + be1[...][None, :], 0.0)
  h2 = jnp.dot(h1, W2[...], preferred_element_type=f32)
  h2 = jnp.maximum((h2 + b2[...][None, :]) * s2[...][None, :]
                   + be2[...][None, :], 0.0)
  deep = jnp.dot(h2, Wout[...], preferred_element_type=f32) + bout[...][None, :]
  out_ref[...] = lin + fm + deep


@jax.jit
def _tc_dense(user_e, item_e, hist_i_sum, hist_g_sum, user_l, item_l,
              hist_i_linp, hist_g_linp, hm_ids, hg_ids, stat_ids, nums,
              stat_emb_W, stat_lin_W, W1, b1, s1, be1, W2, b2, s2, be2,
              Wout, bout):
  return pl.pallas_call(
      _tc_dense_kernel,
      out_shape=jax.ShapeDtypeStruct((B, 1), jnp.float32),
  )(user_e, item_e, hist_i_sum, hist_g_sum, user_l, item_l,
    hist_i_linp, hist_g_linp, hm_ids, hg_ids, stat_ids, nums,
    stat_emb_W, stat_lin_W, W1, b1, s1, be1, W2, b2, s2, be2, Wout, bout)


def kernel(user_idx, item_idx, year_bucket, item_rating_bucket,
           item_count_bucket, user_rating_bucket, user_count_bucket,
           hist_movie_ids, hist_genre_ids, genre_density, genre_recent_match,
           genre_rating_bias, item_emb_W, genre_emb_W, user_emb_W, stat_emb_W,
           item_lin_W, genre_lin_W, user_lin_W, stat_lin_W,
           W1, b1, g1, be1, W2, b2, g2, be2, Wout, bout):
  i32 = jnp.int32
  user_idx = user_idx.astype(i32)
  item_idx = item_idx.astype(i32)
  hm_ids = hist_movie_ids.astype(i32)
  hg_ids = hist_genre_ids.astype(i32)
  hm2 = jnp.pad(hm_ids.reshape(RI, 100), ((0, 0), (0, 12)))
  hg2 = jnp.pad(hg_ids.reshape(RG, 100), ((0, 0), (0, 12)))

  (user_e, item_e, user_l, item_l, hist_i_sum, hist_g_sum,
   hist_i_linp, hist_g_linp) = _sc_gather(
      user_idx, item_idx, hm2, hg2, user_emb_W, item_emb_W, genre_emb_W,
      user_lin_W.reshape(-1), item_lin_W.reshape(-1), genre_lin_W.reshape(-1))

  stat_ids = jnp.stack([year_bucket, item_rating_bucket, item_count_bucket,
                        user_rating_bucket, user_count_bucket],
                       axis=1).astype(i32)
  nums = jnp.stack([genre_density, genre_recent_match, genre_rating_bias],
                   axis=1)
  inv = 1.0 / jnp.sqrt(1.0 + 1e-05)
  out = _tc_dense(
      user_e, item_e, hist_i_sum, hist_g_sum,
      user_l.reshape(B, 1), item_l.reshape(B, 1),
      hist_i_linp, hist_g_linp,
      hm_ids, hg_ids, stat_ids, nums,
      stat_emb_W, stat_lin_W.reshape(5, 100, 1),
      W1, b1, g1 * inv, be1, W2, b2, g2 * inv, be2, Wout, bout)
  return out.reshape(B)


# speedup vs baseline: 1.7054x; 1.7054x over previous
import functools

import jax
import jax.numpy as jnp
from jax import lax
from jax.experimental import pallas as pl
from jax.experimental.pallas import tpu as pltpu
from jax.experimental.pallas import tpu_sc as plsc

B = 4096
D = 16
L_I = 200
L_G = 50
NW = 32
BW = B // NW
RI = 2 * B
RG = B // 2


def _sc_gather_kernel(user_idx, item_idx, hm2, hg2,
                      user_emb, item_emb, genre_emb,
                      user_linf, item_linf, genre_linf,
                      user_e, item_e, user_l, item_l,
                      hist_i_sum, hist_g_sum, hist_i_linv, hist_g_linv,
                      idx_v, rows128_v, lin128_v, idsA_v, idsB_v,
                      growsA_v, growsB_v, linA_v, linB_v,
                      sums_v, sem, semA, semB):
  wid = lax.axis_index("s") * 2 + lax.axis_index("c")
  base = wid * BW

  pltpu.sync_copy(user_idx.at[pl.ds(base, BW)], idx_v)
  pltpu.async_copy(user_emb.at[idx_v], rows128_v, sem).wait()
  pltpu.sync_copy(rows128_v, user_e.at[pl.ds(base, BW)])
  pltpu.async_copy(user_linf.at[idx_v], lin128_v, sem).wait()
  pltpu.sync_copy(lin128_v, user_l.at[pl.ds(base, BW)])

  pltpu.sync_copy(item_idx.at[pl.ds(base, BW)], idx_v)
  pltpu.async_copy(item_emb.at[idx_v], rows128_v, sem).wait()
  pltpu.sync_copy(rows128_v, item_e.at[pl.ds(base, BW)])
  pltpu.async_copy(item_linf.at[idx_v], lin128_v, sem).wait()
  pltpu.sync_copy(lin128_v, item_l.at[pl.ds(base, BW)])

  z16 = jnp.zeros((16,), jnp.float32)

  def hist_pass(ids2_hbm, table, linf, sum_out, linv_out, n_chunks, run_len,
                elems_per_chunk, row_base):
    def copies(ids_s, grows_s, lin_s, sem_s):
      cs = []
      for j in range(16):
        cs.append(pltpu.make_async_copy(
            table.at[ids_s.at[j]], grows_s.at[pl.ds(j * 100, 100)], sem_s))
        cs.append(pltpu.make_async_copy(
            linf.at[ids_s.at[j]], lin_s.at[j], sem_s))
      return cs

    def fire(ch, ids_s, grows_s, lin_s, sem_s):
      pltpu.sync_copy(ids2_hbm.at[pl.ds(row_base + ch * 16, 16)], ids_s)
      for c in copies(ids_s, grows_s, lin_s, sem_s):
        c.start()

    def drain(ids_s, grows_s, lin_s, sem_s):
      for c in copies(ids_s, grows_s, lin_s, sem_s):
        c.wait()

    def process(ch, grows_s, lin_s):
      pltpu.sync_copy(lin_s, linv_out.at[pl.ds(row_base + ch * 16, 16)])
      n4 = run_len // 4
      rem = run_len - n4 * 4
      def elem_body(e, _):
        start = e * run_len
        def add_body(l, accs):
          b = start + 4 * l
          return (accs[0] + grows_s[b], accs[1] + grows_s[b + 1],
                  accs[2] + grows_s[b + 2], accs[3] + grows_s[b + 3])
        a = lax.fori_loop(0, n4, add_body, (z16, z16, z16, z16), unroll=4)
        acc = (a[0] + a[1]) + (a[2] + a[3])
        for r in range(rem):
          acc = acc + grows_s[start + n4 * 4 + r]
        sums_v[ch * elems_per_chunk + e] = acc
        return 0
      lax.fori_loop(0, elems_per_chunk, elem_body, 0)

    fire(0, idsA_v, growsA_v, linA_v, semA)
    def pair_body(p, _):
      ch0 = 2 * p
      drain(idsA_v, growsA_v, linA_v, semA)
      fire(ch0 + 1, idsB_v, growsB_v, linB_v, semB)
      process(ch0, growsA_v, linA_v)
      drain(idsB_v, growsB_v, linB_v, semB)
      @pl.when(ch0 + 2 < n_chunks)
      def _():
        fire(ch0 + 2, idsA_v, growsA_v, linA_v, semA)
      process(ch0 + 1, growsB_v, linB_v)
      return 0
    lax.fori_loop(0, n_chunks // 2, pair_body, 0)
    pltpu.sync_copy(sums_v, sum_out.at[pl.ds(base, BW)])

  hist_pass(hm2, item_emb, item_linf, hist_i_sum, hist_i_linv,
            n_chunks=16, run_len=L_I, elems_per_chunk=8, row_base=wid * 256)
  hist_pass(hg2, genre_emb, genre_linf, hist_g_sum, hist_g_linv,
            n_chunks=4, run_len=L_G, elems_per_chunk=32, row_base=wid * 64)


@jax.jit
def _sc_gather(user_idx, item_idx, hm2, hg2, user_emb, item_emb, genre_emb,
               user_linf, item_linf, genre_linf):
  f32 = jnp.float32
  out_type = (
      jax.ShapeDtypeStruct((B, D), f32),
      jax.ShapeDtypeStruct((B, D), f32),
      jax.ShapeDtypeStruct((B,), f32),
      jax.ShapeDtypeStruct((B,), f32),
      jax.ShapeDtypeStruct((B, D), f32),
      jax.ShapeDtypeStruct((B, D), f32),
      jax.ShapeDtypeStruct((RI, 100), f32),
      jax.ShapeDtypeStruct((RG, 100), f32),
  )
  scratch = [
      pltpu.VMEM((BW,), jnp.int32),
      pltpu.VMEM((BW, D), f32),
      pltpu.VMEM((BW,), f32),
      pltpu.VMEM((16, 100), jnp.int32),
      pltpu.VMEM((16, 100), jnp.int32),
      pltpu.VMEM((1600, D), f32),
      pltpu.VMEM((1600, D), f32),
      pltpu.VMEM((16, 100), f32),
      pltpu.VMEM((16, 100), f32),
      pltpu.VMEM((BW, D), f32),
      pltpu.SemaphoreType.DMA,
      pltpu.SemaphoreType.DMA,
      pltpu.SemaphoreType.DMA,
  ]
  mesh = plsc.VectorSubcoreMesh(core_axis_name="c", subcore_axis_name="s")
  return pl.kernel(_sc_gather_kernel, out_type=out_type, mesh=mesh,
                   scratch_types=scratch,
                   compiler_params=pltpu.CompilerParams(
                       use_tc_tiling_on_sc=False))(
      user_idx, item_idx, hm2, hg2, user_emb, item_emb, genre_emb,
      user_linf, item_linf, genre_linf)


def _tc_dense_kernel(user_e, item_e, hist_i_sum, hist_g_sum,
                     user_l, item_l, hist_i_linp, hist_g_linp,
                     hm_ids, hg_ids, stat_ids, nums,
                     stat_emb_W, stat_lin_W,
                     W1, b1, s1, be1, W2, b2, s2, be2, Wout, bout,
                     out_ref):
  f32 = jnp.float32
  cnt_i = jnp.sum((hm_ids[...] != 0).astype(f32), axis=1, keepdims=True)
  cnt_g = jnp.sum((hg_ids[...] != 0).astype(f32), axis=1, keepdims=True)
  inv_i = 1.0 / (cnt_i + 1e-8)
  inv_g = 1.0 / (cnt_g + 1e-8)

  mean_i = hist_i_sum[...] * inv_i
  mean_g = hist_g_sum[...] * inv_g

  lin = (user_l[...] + item_l[...]
         + jnp.sum(hist_i_linp[...], axis=1, keepdims=True) * inv_i
         + jnp.sum(hist_g_linp[...], axis=1, keepdims=True) * inv_g)

  iota100 = lax.broadcasted_iota(jnp.int32, (B, 100), 1)
  sids = stat_ids[...]
  embs = [user_e[...], item_e[...]]
  for j in range(5):
    oh = (iota100 == sids[:, j][:, None]).astype(f32)
    embs.append(jnp.dot(oh, stat_emb_W[j], preferred_element_type=f32))
    lin = lin + jnp.dot(oh, stat_lin_W[j], preferred_element_type=f32)
  embs.append(mean_i)
  embs.append(mean_g)

  S = embs[0]
  Q = embs[0] * embs[0]
  for e in embs[1:]:
    S = S + e
    Q = Q + e * e
  fm = 0.5 * jnp.sum(S * S - Q, axis=1, keepdims=True)

  h1 = jnp.zeros((B, 128), f32)
  for f in range(9):
    h1 = h1 + jnp.dot(embs[f], W1[f * 16:(f + 1) * 16, :],
                      preferred_element_type=f32)
  nm = nums[...]
  for t in range(3):
    h1 = h1 + nm[:, t][:, None] * W1[144 + t, :][None, :]
  h1 = jnp.maximum((h1 + b1[...][None, :]) * s1[...][None, :]
                   + be1[...][None, :], 0.0)
  h2 = jnp.dot(h1, W2[...], preferred_element_type=f32)
  h2 = jnp.maximum((h2 + b2[...][None, :]) * s2[...][None, :]
                   + be2[...][None, :], 0.0)
  deep = jnp.dot(h2, Wout[...], preferred_element_type=f32) + bout[...][None, :]
  out_ref[...] = lin + fm + deep


@jax.jit
def _tc_dense(user_e, item_e, hist_i_sum, hist_g_sum, user_l, item_l,
              hist_i_linp, hist_g_linp, hm_ids, hg_ids, stat_ids, nums,
              stat_emb_W, stat_lin_W, W1, b1, s1, be1, W2, b2, s2, be2,
              Wout, bout):
  return pl.pallas_call(
      _tc_dense_kernel,
      out_shape=jax.ShapeDtypeStruct((B, 1), jnp.float32),
  )(user_e, item_e, hist_i_sum, hist_g_sum, user_l, item_l,
    hist_i_linp, hist_g_linp, hm_ids, hg_ids, stat_ids, nums,
    stat_emb_W, stat_lin_W, W1, b1, s1, be1, W2, b2, s2, be2, Wout, bout)


def kernel(user_idx, item_idx, year_bucket, item_rating_bucket,
           item_count_bucket, user_rating_bucket, user_count_bucket,
           hist_movie_ids, hist_genre_ids, genre_density, genre_recent_match,
           genre_rating_bias, item_emb_W, genre_emb_W, user_emb_W, stat_emb_W,
           item_lin_W, genre_lin_W, user_lin_W, stat_lin_W,
           W1, b1, g1, be1, W2, b2, g2, be2, Wout, bout):
  i32 = jnp.int32
  user_idx = user_idx.astype(i32)
  item_idx = item_idx.astype(i32)
  hm_ids = hist_movie_ids.astype(i32)
  hg_ids = hist_genre_ids.astype(i32)
  hm2 = hm_ids.reshape(RI, 100)
  hg2 = hg_ids.reshape(RG, 100)

  (user_e, item_e, user_l, item_l, hist_i_sum, hist_g_sum,
   hist_i_linv, hist_g_linv) = _sc_gather(
      user_idx, item_idx, hm2, hg2, user_emb_W, item_emb_W, genre_emb_W,
      user_lin_W.reshape(-1), item_lin_W.reshape(-1), genre_lin_W.reshape(-1))

  stat_ids = jnp.stack([year_bucket, item_rating_bucket, item_count_bucket,
                        user_rating_bucket, user_count_bucket],
                       axis=1).astype(i32)
  nums = jnp.stack([genre_density, genre_recent_match, genre_rating_bias],
                   axis=1)
  inv = 1.0 / jnp.sqrt(1.0 + 1e-05)
  out = _tc_dense(
      user_e, item_e, hist_i_sum, hist_g_sum,
      user_l.reshape(B, 1), item_l.reshape(B, 1),
      hist_i_linv.reshape(B, L_I), hist_g_linv.reshape(B, L_G),
      hm_ids, hg_ids, stat_ids, nums,
      stat_emb_W, stat_lin_W.reshape(5, 100, 1),
      W1, b1, g1 * inv, be1, W2, b2, g2 * inv, be2, Wout, bout)
  return out.reshape(B)
